# baseline (speedup 1.0000x reference)
import functools

import jax
import jax.numpy as jnp
from jax import lax
from jax.experimental import pallas as pl
from jax.experimental.pallas import tpu as pltpu
from jax.experimental.pallas import tpu_sc as plsc

B = 16384
F = 16
N = 1000000
NC = 2
NS = 16
L = 16
NW = NC * NS
BPW = B // NW
BLOCKS = BPW // L

NWIN = N // 128
NT = (NWIN + 1) * 16
RING = 7
ROUNDS = 35

_mesh = plsc.VectorSubcoreMesh(core_axis_name="c", subcore_axis_name="s")


@functools.partial(
    pl.kernel,
    out_type=[
        jax.ShapeDtypeStruct((NT, 128), jnp.float32),
        jax.ShapeDtypeStruct((NT, 128), jnp.float32),
    ],
    mesh=_mesh,
    scratch_types=[
        pltpu.VMEM((3 * RING, 16, 128), jnp.float32),
        pltpu.SemaphoreType.DMA,
        pltpu.SemaphoreType.DMA,
    ],
    compiler_params=pltpu.CompilerParams(use_tc_tiling_on_sc=True),
)
def _relayout(uft_h, ift_h, uit_h, tu_h, ti_h,
              u_out, i_out, bufs, sem_in, sem_out):
    wid = lax.axis_index("s") * NC + lax.axis_index("c")

    tables = (uft_h, uit_h, ift_h)

    def round_(i, carry):
        ks = []
        ins = []
        for j in range(RING):
            k = jnp.minimum((i * RING + j) * NW + wid, NWIN - 1)
            ks.append(k)
            c0 = pl.multiple_of(k * 128, 128)
            for t, tbl in enumerate(tables):
                ins.append(pltpu.async_copy(
                    tbl.at[:, pl.ds(c0, 128)], bufs.at[3 * j + t], sem_in))
        outs = []
        for j in range(RING):
            k = ks[j]
            ins[3 * j].wait()
            ins[3 * j + 1].wait()
            bu = bufs.at[3 * j]
            bi = bufs.at[3 * j + 1]
            for r in range(16):
                for c in range(0, 128, L):
                    sl = pl.ds(c, L)
                    bu[r, sl] = bu[r, sl] + bi[r, sl]
            outs.append(pltpu.async_copy(
                bu, u_out.at[pl.ds(k * 16, 16), :], sem_out))
            ins[3 * j + 2].wait()
            outs.append(pltpu.async_copy(
                bufs.at[3 * j + 2], i_out.at[pl.ds(k * 16, 16), :], sem_out))
        for c in outs:
            c.wait()
        return carry

    lax.fori_loop(0, ROUNDS, round_, 0)

    @pl.when(wid == 0)
    def _():
        pltpu.sync_copy(tu_h, u_out.at[pl.ds(NWIN * 16, 16), :])
        pltpu.sync_copy(ti_h, i_out.at[pl.ds(NWIN * 16, 16), :])


@functools.partial(
    pl.kernel,
    out_type=jax.ShapeDtypeStruct((B,), jnp.float32),
    mesh=_mesh,
    scratch_types=[
        pltpu.VMEM((BPW,), jnp.int32),
        pltpu.VMEM((BPW,), jnp.int32),
        pltpu.VMEM((BPW,), jnp.int32),
        pltpu.VMEM((BPW,), jnp.int32),
        pltpu.VMEM((F, BPW), jnp.int32),
        pltpu.VMEM((F, BPW), jnp.int32),
        pltpu.VMEM((F, BPW), jnp.float32),
        pltpu.VMEM((F, BPW), jnp.float32),
        pltpu.VMEM((BPW,), jnp.float32),
        pltpu.VMEM((BPW,), jnp.float32),
        pltpu.VMEM((BPW,), jnp.float32),
        pltpu.SemaphoreType.DMA,
    ],
    compiler_params=pltpu.CompilerParams(
        use_tc_tiling_on_sc=False, needs_layout_passes=False),
)
def _svdpp(user_h, item_h, u_lin, i_lin, ub_h, ib_h, out_h,
           idx_u, idx_i, base_u, base_i, widx_u, widx_i,
           uf_v, if_v, ub_v, ib_v, out_v, sem):
    wid = lax.axis_index("s") * NC + lax.axis_index("c")
    base = wid * BPW

    pltpu.sync_copy(user_h.at[pl.ds(base, BPW)], idx_u)
    pltpu.sync_copy(item_h.at[pl.ds(base, BPW)], idx_i)

    def mk_base(j, carry):
        sl = pl.ds(j * L, L)
        vu = idx_u[sl]
        vi = idx_i[sl]
        base_u[sl] = (vu >> 7) * 2048 + (vu & 127)
        base_i[sl] = (vi >> 7) * 2048 + (vi & 127)
        return carry

    lax.fori_loop(0, BLOCKS, mk_base, 0)

    for f in range(F):
        def mk_widx(j, carry, f=f):
            sl = pl.ds(j * L, L)
            widx_u.at[f][sl] = base_u[sl] + (f * 128)
            widx_i.at[f][sl] = base_i[sl] + (f * 128)
            return carry
        lax.fori_loop(0, BLOCKS, mk_widx, 0)

    copies = []
    for f in range(F):
        copies.append(pltpu.async_copy(u_lin.at[widx_u.at[f]], uf_v.at[f], sem))
        copies.append(pltpu.async_copy(i_lin.at[widx_i.at[f]], if_v.at[f], sem))
    copies.append(pltpu.async_copy(ub_h.at[idx_u], ub_v, sem))
    copies.append(pltpu.async_copy(ib_h.at[idx_i], ib_v, sem))
    for c in copies:
        c.wait()

    def block(blk, carry):
        sl = pl.ds(blk * L, L)
        acc = ub_v[sl] + ib_v[sl]
        for f in range(F):
            acc = acc + uf_v[f, sl] * if_v[f, sl]
        out_v[sl] = 1.0 / (1.0 + jnp.exp(-acc))
        return carry

    lax.fori_loop(0, BLOCKS, block, 0)

    pltpu.sync_copy(out_v, out_h.at[pl.ds(base, BPW)])


def _tail(tbl_t):
    pad = jnp.zeros((16, 128 - (N - NWIN * 128)), jnp.float32)
    return jnp.concatenate([tbl_t[:, NWIN * 128:], pad], axis=1)


def kernel(user, item, user_factors, item_factors, user_biases,
           item_biases, user_implicit):
    uft = user_factors.T
    ift = item_factors.T
    uit = user_implicit.T
    u_lin, i_lin = _relayout(uft, ift, uit,
                             _tail(uft) + _tail(uit), _tail(ift))
    ub = user_biases.reshape((-1,))
    ib = item_biases.reshape((-1,))
    return _svdpp(user, item, u_lin.reshape((-1,)), i_lin.reshape((-1,)),
                  ub, ib)


# speedup vs baseline: 1.2029x; 1.2029x over previous
import functools

import jax
import jax.numpy as jnp
from jax import lax
from jax.experimental import pallas as pl
from jax.experimental.pallas import tpu as pltpu
from jax.experimental.pallas import tpu_sc as plsc

B = 16384
F = 16
N = 1000000
NC = 2
NS = 16
L = 16
NW = NC * NS
BPW = B // NW
BLOCKS = BPW // L

NWIN = N // 128
NT = (NWIN + 1) * 16
RING = 7
ROUNDS = 35

_mesh = plsc.VectorSubcoreMesh(core_axis_name="c", subcore_axis_name="s")


@functools.partial(
    pl.kernel,
    out_type=[
        jax.ShapeDtypeStruct((NT, 128), jnp.float32),
        jax.ShapeDtypeStruct((NT, 128), jnp.float32),
        jax.ShapeDtypeStruct((NT, 128), jnp.float32),
    ],
    mesh=_mesh,
    scratch_types=[
        pltpu.VMEM((3 * RING, 16, 128), jnp.float32),
        pltpu.SemaphoreType.DMA,
        pltpu.SemaphoreType.DMA,
    ],
    compiler_params=pltpu.CompilerParams(use_tc_tiling_on_sc=True),
)
def _relayout(uft_h, ift_h, uit_h, tu_h, ti_h, tui_h,
              u_out, i_out, ui_out, bufs, sem_in, sem_out):
    wid = lax.axis_index("s") * NC + lax.axis_index("c")

    tables = ((uft_h, u_out), (ift_h, i_out), (uit_h, ui_out))

    def round_(i, carry):
        ks = []
        ins = []
        for t, (tbl, _) in enumerate(tables):
            for j in range(RING):
                k = jnp.minimum((i * RING + j) * NW + wid, NWIN - 1)
                ks.append(k)
                c0 = pl.multiple_of(k * 128, 128)
                ins.append(pltpu.async_copy(
                    tbl.at[:, pl.ds(c0, 128)], bufs.at[t * RING + j], sem_in))
        outs = []
        for t, (_, out) in enumerate(tables):
            for j in range(RING):
                k = ks[t * RING + j]
                ins[t * RING + j].wait()
                outs.append(pltpu.async_copy(
                    bufs.at[t * RING + j], out.at[pl.ds(k * 16, 16), :],
                    sem_out))
        for c in outs:
            c.wait()
        return carry

    lax.fori_loop(0, ROUNDS, round_, 0)

    @pl.when(wid == 0)
    def _():
        for tail, out in ((tu_h, u_out), (ti_h, i_out), (tui_h, ui_out)):
            pltpu.sync_copy(tail, out.at[pl.ds(NWIN * 16, 16), :])


@functools.partial(
    pl.kernel,
    out_type=jax.ShapeDtypeStruct((B,), jnp.float32),
    mesh=_mesh,
    scratch_types=[
        pltpu.VMEM((BPW,), jnp.int32),
        pltpu.VMEM((BPW,), jnp.int32),
        pltpu.VMEM((BPW,), jnp.int32),
        pltpu.VMEM((BPW,), jnp.int32),
        pltpu.VMEM((F, BPW), jnp.int32),
        pltpu.VMEM((F, BPW), jnp.int32),
        pltpu.VMEM((F, BPW), jnp.float32),
        pltpu.VMEM((F, BPW), jnp.float32),
        pltpu.VMEM((F, BPW), jnp.float32),
        pltpu.VMEM((BPW,), jnp.float32),
        pltpu.VMEM((BPW,), jnp.float32),
        pltpu.VMEM((BPW,), jnp.float32),
        pltpu.SemaphoreType.DMA,
    ],
    compiler_params=pltpu.CompilerParams(
        use_tc_tiling_on_sc=False, needs_layout_passes=False),
)
def _svdpp(user_h, item_h, u_lin, i_lin, ui_lin, ub_h, ib_h, out_h,
           idx_u, idx_i, base_u, base_i, widx_u, widx_i,
           uf_v, ui_v, if_v, ub_v, ib_v, out_v, sem):
    wid = lax.axis_index("s") * NC + lax.axis_index("c")
    base = wid * BPW

    pltpu.sync_copy(user_h.at[pl.ds(base, BPW)], idx_u)
    pltpu.sync_copy(item_h.at[pl.ds(base, BPW)], idx_i)

    def mk_base(j, carry):
        sl = pl.ds(j * L, L)
        vu = idx_u[sl]
        vi = idx_i[sl]
        base_u[sl] = (vu >> 7) * 2048 + (vu & 127)
        base_i[sl] = (vi >> 7) * 2048 + (vi & 127)
        return carry

    lax.fori_loop(0, BLOCKS, mk_base, 0)

    for f in range(F):
        def mk_widx(j, carry, f=f):
            sl = pl.ds(j * L, L)
            widx_u.at[f][sl] = base_u[sl] + (f * 128)
            widx_i.at[f][sl] = base_i[sl] + (f * 128)
            return carry
        lax.fori_loop(0, BLOCKS, mk_widx, 0)

    copies = []
    for f in range(F):
        copies.append(pltpu.async_copy(u_lin.at[widx_u.at[f]], uf_v.at[f], sem))
        copies.append(pltpu.async_copy(ui_lin.at[widx_u.at[f]], ui_v.at[f], sem))
        copies.append(pltpu.async_copy(i_lin.at[widx_i.at[f]], if_v.at[f], sem))
    copies.append(pltpu.async_copy(ub_h.at[idx_u], ub_v, sem))
    copies.append(pltpu.async_copy(ib_h.at[idx_i], ib_v, sem))
    for c in copies:
        c.wait()

    def block(blk, carry):
        sl = pl.ds(blk * L, L)
        acc = ub_v[sl] + ib_v[sl]
        for f in range(F):
            acc = acc + (uf_v[f, sl] + ui_v[f, sl]) * if_v[f, sl]
        out_v[sl] = 1.0 / (1.0 + jnp.exp(-acc))
        return carry

    lax.fori_loop(0, BLOCKS, block, 0)

    pltpu.sync_copy(out_v, out_h.at[pl.ds(base, BPW)])


def _tail(tbl_t):
    pad = jnp.zeros((16, 128 - (N - NWIN * 128)), jnp.float32)
    return jnp.concatenate([tbl_t[:, NWIN * 128:], pad], axis=1)


def kernel(user, item, user_factors, item_factors, user_biases,
           item_biases, user_implicit):
    uft = user_factors.T
    ift = item_factors.T
    uit = user_implicit.T
    u_lin, i_lin, ui_lin = _relayout(uft, ift, uit,
                                     _tail(uft), _tail(ift), _tail(uit))
    ub = user_biases.reshape((-1,))
    ib = item_biases.reshape((-1,))
    return _svdpp(user, item, u_lin.reshape((-1,)), i_lin.reshape((-1,)),
                  ui_lin.reshape((-1,)), ub, ib)
